# baseline (speedup 1.0000x reference)
import functools

import jax
import jax.numpy as jnp
from jax import lax
from jax.experimental import pallas as pl
from jax.experimental.pallas import tpu as pltpu
from jax.experimental.pallas import tpu_sc as plsc

_ROWS = 8192
_COLS = 3072
_NC = 2
_NS = 16
_NW = _NC * _NS
_RPW = _ROWS // _NW
_RBLK = 8
_NBLK = _RPW // _RBLK
_NITER = _NBLK // 2
_JCH = _COLS // 16


def _shuffle_body(in_hbm, perm_hbm, out_hbm, perm_v,
                  in0, in1, in2, out0, out1,
                  sem_in0, sem_in1, sem_in2, sem_out0, sem_out1):
    c = lax.axis_index("c")
    s = lax.axis_index("s")
    wid = s * _NC + c
    row0 = wid * _RPW

    pltpu.sync_copy(perm_hbm, perm_v)

    in_bufs = (in0, in1, in2)
    out_bufs = (out0, out1)
    in_sems = (sem_in0, sem_in1, sem_in2)
    out_sems = (sem_out0, sem_out1)

    def in_copy(b, ki):
        return pltpu.make_async_copy(
            in_hbm.at[pl.ds(row0 + b * _RBLK, _RBLK), :], in_bufs[ki],
            in_sems[ki])

    def out_copy(b, ko):
        return pltpu.make_async_copy(
            out_bufs[ko], out_hbm.at[pl.ds(row0 + b * _RBLK, _RBLK), :],
            out_sems[ko])

    def gather_block(ki, ko):
        @plsc.parallel_loop(0, _JCH, unroll=4)
        def jloop(j):
            idx = perm_v[pl.ds(j * 16, 16)]
            off = j * 16
            for r in range(_RBLK):
                vals = plsc.load_gather(
                    in_bufs[ki], [jnp.full((16,), r, jnp.int32), idx])
                out_bufs[ko][r, pl.ds(off, 16)] = vals

    in_copy(0, 0).start()
    in_copy(1, 1).start()

    def biter(i, carry):
        for k in range(6):
            b = i * 6 + k
            ki, ko = k % 3, k % 2
            in_copy(b, ki).wait()
            if k >= 2:
                out_copy(b, ko).wait()
            else:
                @pl.when(i > 0)
                def _():
                    out_copy(b, ko).wait()

            gather_block(ki, ko)
            in_copy(b + 2, (k + 2) % 3).start()
            out_copy(b, ko).start()
        return carry

    lax.fori_loop(0, (_NBLK - 2) // 6, biter, 0)

    for b in (_NBLK - 2, _NBLK - 1):
        ki, ko = b % 3, b % 2
        in_copy(b, ki).wait()
        out_copy(b, ko).wait()
        gather_block(ki, ko)
        out_copy(b, ko).start()

    out_copy(_NBLK - 2, _NBLK % 2).wait()
    out_copy(_NBLK - 1, (_NBLK + 1) % 2).wait()


_shuffle = pl.kernel(
    _shuffle_body,
    out_type=jax.ShapeDtypeStruct((_ROWS, _COLS), jnp.float32),
    mesh=plsc.VectorSubcoreMesh(core_axis_name="c", subcore_axis_name="s"),
    scratch_types=[
        pltpu.VMEM((_COLS,), jnp.int32),
        pltpu.VMEM((_RBLK, _COLS), jnp.float32),
        pltpu.VMEM((_RBLK, _COLS), jnp.float32),
        pltpu.VMEM((_RBLK, _COLS), jnp.float32),
        pltpu.VMEM((_RBLK, _COLS), jnp.float32),
        pltpu.VMEM((_RBLK, _COLS), jnp.float32),
        pltpu.SemaphoreType.DMA,
        pltpu.SemaphoreType.DMA,
        pltpu.SemaphoreType.DMA,
        pltpu.SemaphoreType.DMA,
        pltpu.SemaphoreType.DMA,
    ],
    compiler_params=pltpu.CompilerParams(needs_layout_passes=False),
)


def kernel(inputs, perm):
    perm_i = perm.astype(jnp.int32)
    out = _shuffle(inputs, perm_i)
    logdet = jnp.zeros((inputs.shape[0], 1), dtype=jnp.float32)
    return out, logdet


# speedup vs baseline: 1.0305x; 1.0305x over previous
import functools

import jax
import jax.numpy as jnp
from jax import lax
from jax.experimental import pallas as pl
from jax.experimental.pallas import tpu as pltpu
from jax.experimental.pallas import tpu_sc as plsc

_ROWS = 8192
_COLS = 3072
_NC = 2
_NS = 16
_NW = _NC * _NS
_RPW = _ROWS // _NW
_RBLK = 8
_NBLK = _RPW // _RBLK
_NITER = _NBLK // 2
_JCH = _COLS // 16


def _shuffle_body(in_hbm, perm_hbm, out_hbm, perm_v,
                  in0, in1, out0, out1,
                  sem_in0, sem_in1, sem_out0, sem_out1):
    c = lax.axis_index("c")
    s = lax.axis_index("s")
    wid = s * _NC + c
    row0 = wid * _RPW

    in_bufs = (in0, in1)
    out_bufs = (out0, out1)
    in_sems = (sem_in0, sem_in1)
    out_sems = (sem_out0, sem_out1)

    def in_copy(b, ki):
        return pltpu.make_async_copy(
            in_hbm.at[pl.ds(row0 + b * _RBLK, _RBLK), :], in_bufs[ki],
            in_sems[ki])

    def out_copy(b, ko):
        return pltpu.make_async_copy(
            out_bufs[ko], out_hbm.at[pl.ds(row0 + b * _RBLK, _RBLK), :],
            out_sems[ko])

    def gather_block(k):
        @plsc.parallel_loop(0, _JCH, unroll=4)
        def jloop(j):
            idx = perm_v[pl.ds(j * 16, 16)]
            off = j * 16
            for r in range(_RBLK):
                vals = plsc.load_gather(
                    in_bufs[k], [jnp.full((16,), r, jnp.int32), idx])
                out_bufs[k][r, pl.ds(off, 16)] = vals

    in_copy(0, 0).start()
    in_copy(1, 1).start()
    pltpu.sync_copy(perm_hbm, perm_v)

    def biter(i, carry):
        for k in range(2):
            b = i * 2 + k
            in_copy(b, k).wait()

            @pl.when(i > 0)
            def _():
                out_copy(b, k).wait()

            gather_block(k)

            @pl.when(i < _NITER - 1)
            def _():
                in_copy(b + 2, k).start()

            out_copy(b, k).start()
        return carry

    lax.fori_loop(0, _NITER, biter, 0)

    out_copy(_NBLK - 2, 0).wait()
    out_copy(_NBLK - 1, 1).wait()


_shuffle = pl.kernel(
    _shuffle_body,
    out_type=jax.ShapeDtypeStruct((_ROWS, _COLS), jnp.float32),
    mesh=plsc.VectorSubcoreMesh(core_axis_name="c", subcore_axis_name="s"),
    scratch_types=[
        pltpu.VMEM((_COLS,), jnp.int32),
        pltpu.VMEM((_RBLK, _COLS), jnp.float32),
        pltpu.VMEM((_RBLK, _COLS), jnp.float32),
        pltpu.VMEM((_RBLK, _COLS), jnp.float32),
        pltpu.VMEM((_RBLK, _COLS), jnp.float32),
        pltpu.SemaphoreType.DMA,
        pltpu.SemaphoreType.DMA,
        pltpu.SemaphoreType.DMA,
        pltpu.SemaphoreType.DMA,
    ],
    compiler_params=pltpu.CompilerParams(needs_layout_passes=False),
)


def kernel(inputs, perm):
    perm_i = perm.astype(jnp.int32)
    out = _shuffle(inputs, perm_i)
    logdet = jnp.zeros((inputs.shape[0], 1), dtype=jnp.float32)
    return out, logdet
